# Initial kernel scaffold; baseline (speedup 1.0000x reference)
import jax
import jax.numpy as jnp
from jax.experimental import pallas as pl


def kernel(vertices, unpool_idx):
    raise NotImplementedError("write your pallas kernel here")


# speedup vs baseline: 4.1693x; 4.1693x over previous
import functools
import jax
import jax.numpy as jnp
from jax import lax
from jax.experimental import pallas as pl
from jax.experimental.pallas import tpu as pltpu
from jax.experimental.pallas import tpu_sc as plsc

B, NV, NE, D = 4, 10000, 160000, 128
NC, NS, L = 2, 16, 16
NW = NC * NS
K = 128
NCHUNK = NE // K
CBASE, CREM = NCHUNK // NW, NCHUNK % NW
CP_ROWS = 80
CP_PER_B = NV // CP_ROWS
NCOPY = B * CP_PER_B
CPBASE, CPREM = NCOPY // NW, NCOPY % NW

_mesh = plsc.VectorSubcoreMesh(core_axis_name="c", subcore_axis_name="s")


@functools.partial(
    pl.kernel,
    out_type=jax.ShapeDtypeStruct((B, NV + NE, D), jnp.float32),
    mesh=_mesh,
    scratch_types=[
        pltpu.VMEM((K,), jnp.int32),
        pltpu.VMEM((K,), jnp.int32),
        pltpu.VMEM((K,), jnp.int32),
        pltpu.VMEM((K,), jnp.int32),
        pltpu.VMEM((K, D), jnp.float32),
        pltpu.VMEM((K, D), jnp.float32),
        pltpu.VMEM((K, D), jnp.float32),
        pltpu.VMEM((CP_ROWS, D), jnp.float32),
        pltpu.SemaphoreType.DMA,
        pltpu.SemaphoreType.DMA,
    ],
)
def _unpool_kernel(vflat, i0, i1, out, idx0_v, idx1_v, idxb0, idxb1,
                   rowsA, rowsB, obuf, cbuf, sema, semb):
    wid = lax.axis_index("s") * NC + lax.axis_index("c")

    cp_cnt = CPBASE + jnp.where(wid < CPREM, 1, 0)
    cp_lo = wid * CPBASE + jnp.minimum(wid, CPREM)

    def copy_body(t, carry):
        q = cp_lo + t
        b = q // CP_PER_B
        r0 = (q % CP_PER_B) * CP_ROWS
        pltpu.sync_copy(vflat.at[pl.ds(b * NV + r0, CP_ROWS)], cbuf)
        pltpu.sync_copy(cbuf, out.at[b, pl.ds(r0, CP_ROWS)])
        return carry

    lax.fori_loop(0, cp_cnt, copy_body, 0)

    cnt = CBASE + jnp.where(wid < CREM, 1, 0)
    lo = wid * CBASE + jnp.minimum(wid, CREM)

    def chunk_body(t, carry):
        c = lo + t
        e0 = c * K
        pltpu.sync_copy(i0.at[pl.ds(e0, K)], idx0_v)
        pltpu.sync_copy(i1.at[pl.ds(e0, K)], idx1_v)
        for b in range(B):
            off = b * NV
            for j in range(K // L):
                sl = pl.ds(j * L, L)
                idxb0[sl] = idx0_v[sl] + off
                idxb1[sl] = idx1_v[sl] + off
            ga = pltpu.async_copy(vflat.at[idxb0], rowsA, sema)
            gb = pltpu.async_copy(vflat.at[idxb1], rowsB, semb)
            ga.wait()
            gb.wait()

            def row_body(r, rcarry):
                for j in range(D // L):
                    sl = pl.ds(j * L, L)
                    obuf[r, sl] = (rowsA[r, sl] + rowsB[r, sl]) * 0.5
                return rcarry

            lax.fori_loop(0, K, row_body, 0)
            pltpu.sync_copy(obuf, out.at[b, pl.ds(NV + e0, K)])
        return carry

    lax.fori_loop(0, cnt, chunk_body, 0)


def kernel(vertices, unpool_idx):
    vflat = vertices.reshape(B * NV, D)
    i0 = unpool_idx[:, 0]
    i1 = unpool_idx[:, 1]
    return _unpool_kernel(vflat, i0, i1)


# speedup vs baseline: 4.5632x; 1.0945x over previous
import functools
import jax
import jax.numpy as jnp
from jax import lax
from jax.experimental import pallas as pl
from jax.experimental.pallas import tpu as pltpu
from jax.experimental.pallas import tpu_sc as plsc

B, NV, NE, D = 4, 10000, 160000, 128
NC, NS, L = 2, 16, 16
NW = NC * NS
K = 128
NCHUNK = NE // K
CBASE, CREM = NCHUNK // NW, NCHUNK % NW
CP_ROWS = 1248

_mesh = plsc.VectorSubcoreMesh(core_axis_name="c", subcore_axis_name="s")


@functools.partial(
    pl.kernel,
    out_type=jax.ShapeDtypeStruct((B, NV + NE, D), jnp.float32),
    mesh=_mesh,
    scratch_types=[
        pltpu.VMEM((K,), jnp.int32),
        pltpu.VMEM((K,), jnp.int32),
        pltpu.VMEM((K,), jnp.int32),
        pltpu.VMEM((K,), jnp.int32),
        pltpu.VMEM((K,), jnp.int32),
        pltpu.VMEM((K,), jnp.int32),
        pltpu.VMEM((K, D), jnp.float32),
        pltpu.VMEM((K, D), jnp.float32),
        pltpu.VMEM((K, D), jnp.float32),
        pltpu.VMEM((K, D), jnp.float32),
        pltpu.VMEM((K, D), jnp.float32),
        pltpu.VMEM((K, D), jnp.float32),
        pltpu.SemaphoreType.DMA,
        pltpu.SemaphoreType.DMA,
        pltpu.SemaphoreType.DMA,
        pltpu.SemaphoreType.DMA,
        pltpu.SemaphoreType.DMA,
    ],
)
def _unpool_kernel(vflat, i0, i1, out,
                   idx0_v, idx1_v, ib0_0, ib0_1, ib1_0, ib1_1,
                   rA0, rA1, rB0, rB1, ob0, ob1,
                   sg0, sg1, sw0, sw1, sc):
    idxb0 = [ib0_0, ib0_1]
    idxb1 = [ib1_0, ib1_1]
    rowsA = [rA0, rA1]
    rowsB = [rB0, rB1]
    obuf = [ob0, ob1]
    semG = [sg0, sg1]
    semW = [sw0, sw1]

    wid = lax.axis_index("s") * NC + lax.axis_index("c")

    cb = wid // 8
    cr0 = (wid % 8) * CP_ROWS
    cp = pltpu.async_copy(vflat.at[pl.ds(cb * NV + cr0, CP_ROWS)],
                          out.at[cb, pl.ds(cr0, CP_ROWS)], sc)
    RREM = NV - 8 * CP_ROWS

    @pl.when(wid < B)
    def _():
        pltpu.async_copy(vflat.at[pl.ds(wid * NV + 8 * CP_ROWS, RREM)],
                         out.at[wid, pl.ds(8 * CP_ROWS, RREM)], sc)

    cnt = CBASE + jnp.where(wid < CREM, 1, 0).astype(jnp.int32)
    lo = wid * CBASE + jnp.minimum(wid, CREM)

    def load_idx(c):
        pltpu.sync_copy(i0.at[pl.ds(c * K, K)], idx0_v)
        pltpu.sync_copy(i1.at[pl.ds(c * K, K)], idx1_v)

    def adjust(p, off):
        for j in range(K // L):
            sl = pl.ds(j * L, L)
            idxb0[p][sl] = idx0_v[sl] + off
            idxb1[p][sl] = idx1_v[sl] + off

    def fire_gather(p):
        pltpu.async_copy(vflat.at[idxb0[p]], rowsA[p], semG[p])
        pltpu.async_copy(vflat.at[idxb1[p]], rowsB[p], semG[p])

    def wait_gather(p):
        pltpu.make_async_copy(vflat.at[idxb0[p]], rowsA[p], semG[p]).wait()
        pltpu.make_async_copy(vflat.at[idxb1[p]], rowsB[p], semG[p]).wait()

    def wait_write(p):
        pltpu.make_async_copy(obuf[p], out.at[0, pl.ds(NV, K)], semW[p]).wait()

    load_idx(lo)
    adjust(0, 0)
    fire_gather(0)

    def chunk_body(t, carry):
        c = lo + t
        for b in range(B):
            p = b & 1
            q = p ^ 1
            if b < B - 1:
                adjust(q, (b + 1) * NV)
                fire_gather(q)
            else:
                @pl.when(t + 1 < cnt)
                def _():
                    load_idx(c + 1)
                    adjust(q, 0)
                    fire_gather(q)
            wait_gather(p)
            if b >= 2:
                wait_write(p)
            else:
                @pl.when(t > 0)
                def _():
                    wait_write(p)
            def row_body(r, rcarry):
                for j in range(D // L):
                    sl = pl.ds(j * L, L)
                    obuf[p][r, sl] = (rowsA[p][r, sl] + rowsB[p][r, sl]) * 0.5
                return rcarry

            lax.fori_loop(0, K, row_body, 0)
            pltpu.async_copy(obuf[p], out.at[b, pl.ds(NV + c * K, K)], semW[p])
        return carry

    lax.fori_loop(0, cnt, chunk_body, 0)

    wait_write(0)
    wait_write(1)
    cp.wait()

    @pl.when(wid < B)
    def _():
        pltpu.make_async_copy(vflat.at[pl.ds(wid * NV + 8 * CP_ROWS, RREM)],
                              out.at[wid, pl.ds(8 * CP_ROWS, RREM)], sc).wait()


def kernel(vertices, unpool_idx):
    vflat = vertices.reshape(B * NV, D)
    i0 = unpool_idx[:, 0]
    i1 = unpool_idx[:, 1]
    return _unpool_kernel(vflat, i0, i1)
